# baseline (speedup 1.0000x reference)
import functools

import jax
import jax.numpy as jnp
from jax import lax
from jax.experimental import pallas as pl
from jax.experimental.pallas import tpu as pltpu
from jax.experimental.pallas import tpu_sc as plsc

VOCAB = 1000000
EMBED = 64
EMBED_PAD = 128
NUM_CLASSES = 16
BATCH = 4096
SEQ = 200

NC = 2
NS = 16
NW = NC * NS
B_PER_W = BATCH // NW
LANES = 16
QV = EMBED // LANES
UNROLL = 8
SEQ_PAD = 256
IDX_PER_W = B_PER_W * SEQ_PAD
PW_PER_ROW = 8
PW_PER_W = B_PER_W * PW_PER_ROW


def _pool_body(xp_hbm, pw_hbm, tbl_hbm, out_hbm, idx_v, pw_v, rows_v,
               pooled_v, sem0, sem1):
    wid = lax.axis_index("s") * NC + lax.axis_index("c")
    base = wid * B_PER_W

    pltpu.sync_copy(xp_hbm.at[pl.ds(wid * IDX_PER_W, IDX_PER_W)], idx_v)
    pltpu.sync_copy(
        pw_hbm.at[pl.ds(wid * PW_PER_W, PW_PER_W)],
        pw_v.at[pl.ds(0, PW_PER_W)],
    )

    sems = (sem0, sem1)

    def idx_slice(i):
        off = pl.multiple_of(i * SEQ_PAD, SEQ_PAD)
        return idx_v.at[pl.ds(off, SEQ)]

    pltpu.async_copy(tbl_hbm.at[idx_slice(0)], rows_v.at[0], sem0)

    neg_inf = jnp.full((LANES,), -jnp.inf, dtype=jnp.float32)

    def outer(g, carry):
        for b in range(2):
            i = g * 2 + b
            pltpu.make_async_copy(
                tbl_hbm.at[idx_slice(0)], rows_v.at[b], sems[b]
            ).wait()

            @pl.when(i + 1 < B_PER_W)
            def _():
                pltpu.async_copy(
                    tbl_hbm.at[idx_slice(i + 1)], rows_v.at[1 - b],
                    sems[1 - b]
                )

            def red(t, accs):
                a = list(accs)
                j0 = t * UNROLL
                w = pw_v[pl.ds(i * PW_PER_ROW + t // 4, LANES)][0]
                s0 = (t % 4) * UNROLL
                for u in range(UNROLL):
                    colbase = ((w >> (s0 + u)) & 1) * EMBED
                    for q in range(QV):
                        a[q] = jnp.maximum(
                            a[q],
                            rows_v[b, j0 + u,
                                   pl.ds(colbase + q * LANES, LANES)],
                        )
                return tuple(a)

            accs = lax.fori_loop(
                0, SEQ // UNROLL, red, (neg_inf,) * QV, unroll=False
            )
            for q in range(QV):
                pooled_v[i, pl.ds(q * LANES, LANES)] = accs[q]
        return carry

    lax.fori_loop(0, B_PER_W // 2, outer, 0, unroll=False)

    pltpu.sync_copy(pooled_v, out_hbm.at[pl.ds(base, B_PER_W)])


_pool = functools.partial(
    pl.kernel,
    out_type=jax.ShapeDtypeStruct((BATCH, EMBED_PAD), jnp.float32),
    mesh=plsc.VectorSubcoreMesh(core_axis_name="c", subcore_axis_name="s"),
    scratch_types=[
        pltpu.VMEM((IDX_PER_W,), jnp.int32),
        pltpu.VMEM((PW_PER_W + LANES,), jnp.int32),
        pltpu.VMEM((2, SEQ, EMBED_PAD), jnp.float32),
        pltpu.VMEM((B_PER_W, EMBED_PAD), jnp.float32),
        pltpu.SemaphoreType.DMA,
        pltpu.SemaphoreType.DMA,
    ],
)(_pool_body)


def _mlp_body(p_ref, w1_ref, b1_ref, w2_ref, b2_ref, o_ref):
    pooled = p_ref[...][:, :EMBED]
    h = jnp.maximum(
        jnp.dot(pooled, w1_ref[...], preferred_element_type=jnp.float32)
        + b1_ref[...],
        0.0,
    )
    o_ref[...] = (
        jnp.dot(h, w2_ref[...], preferred_element_type=jnp.float32)
        + b2_ref[...]
    )


def kernel(x, table, W1, b1, W2, b2):
    xi = x.astype(jnp.int32)
    xp = jnp.pad(xi >> 1, ((0, 0), (0, SEQ_PAD - SEQ)))
    par = jnp.pad((xi & 1).astype(jnp.uint32), ((0, 0), (0, 224 - SEQ)))
    shifts = jnp.arange(32, dtype=jnp.uint32)
    pw = jnp.sum(par.reshape(BATCH, 7, 32) << shifts, axis=-1,
                 dtype=jnp.uint32)
    pw = jnp.pad(pw, ((0, 0), (0, PW_PER_ROW - 7))).astype(jnp.int32)
    tbl2 = jnp.concatenate([table[0::2, :], table[1::2, :]], axis=1)
    pooled = _pool(xp.reshape(-1), pw.reshape(-1), tbl2)
    out = pl.pallas_call(
        _mlp_body,
        out_shape=jax.ShapeDtypeStruct((BATCH, NUM_CLASSES), jnp.float32),
    )(pooled, W1, b1.reshape(1, EMBED), W2, b2.reshape(1, NUM_CLASSES))
    return out


# speedup vs baseline: 6.1468x; 6.1468x over previous
import functools

import jax
import jax.numpy as jnp
from jax import lax
from jax.experimental import pallas as pl
from jax.experimental.pallas import tpu as pltpu
from jax.experimental.pallas import tpu_sc as plsc

VOCAB = 1000000
EMBED = 64
EMBED_PAD = 128
NUM_CLASSES = 16
BATCH = 4096
SEQ = 200

NC = 2
NS = 16
NW = NC * NS
B_PER_W = BATCH // NW
LANES = 16
QV = EMBED // LANES
UNROLL = 8
SEQ_PAD = 256
IDX_PER_W = B_PER_W * SEQ_PAD
PW_PER_ROW = 8
PW_PER_W = B_PER_W * PW_PER_ROW


W_PACK = 384
N_BLOCKS = VOCAB // W_PACK
N_COMMON = (N_BLOCKS // NW) * NW
N_STEPS = N_COMMON // NW
REM_BASE = N_BLOCKS * W_PACK


def _pack_body(tblT_hbm, tail_hbm, out_hbm, slab_v, dst_v, sin0, sin1, sout0, sout1):
    wid = lax.axis_index("s") * NC + lax.axis_index("c")
    sins = (sin0, sin1)
    souts = (sout0, sout1)
    lane = lax.iota(jnp.int32, LANES)

    def c0_of(s):
        return pl.multiple_of((s * NW + wid) * W_PACK, 128)

    def in_slice(s):
        return tblT_hbm.at[:, pl.ds(c0_of(s), W_PACK)]

    def out_slice(s):
        off = pl.multiple_of((s * NW + wid) * (W_PACK // 2), 8)
        return out_hbm.at[pl.ds(off, W_PACK // 2)]

    def transpose_slab(buf, width):
        def body_d(d, carry):
            def body_t(t, carry2):
                j = t * LANES + lane
                v = slab_v[buf, d, pl.ds(t * LANES, LANES)]
                plsc.store_scatter(
                    dst_v.at[buf], [j >> 1, ((j & 1) << 6) + d], v
                )
                return carry2
            return lax.fori_loop(0, width // LANES, body_t, carry,
                                 unroll=False)
        lax.fori_loop(0, EMBED, body_d, 0, unroll=False)

    pltpu.async_copy(in_slice(0), slab_v.at[0], sin0)

    def step(g, carry):
        for b in range(2):
            ss = g * 2 + b
            pltpu.make_async_copy(in_slice(0), slab_v.at[b], sins[b]).wait()

            @pl.when(ss + 1 < N_STEPS)
            def _():
                pltpu.async_copy(in_slice(ss + 1), slab_v.at[1 - b],
                                 sins[1 - b])

            @pl.when(ss >= 2)
            def _():
                pltpu.make_async_copy(
                    dst_v.at[b], out_slice(0), souts[b]
                ).wait()

            transpose_slab(b, W_PACK)
            pltpu.async_copy(dst_v.at[b], out_slice(ss), souts[b])
        return carry

    lax.fori_loop(0, N_STEPS // 2, step, 0, unroll=False)

    ss = N_STEPS - 1
    b = ss % 2
    pltpu.make_async_copy(in_slice(0), slab_v.at[b], sins[b]).wait()
    pltpu.make_async_copy(dst_v.at[b], out_slice(0), souts[b]).wait()
    transpose_slab(b, W_PACK)
    pltpu.async_copy(dst_v.at[b], out_slice(ss), souts[b])

    pltpu.make_async_copy(dst_v.at[1 - b], out_slice(0), souts[1 - b]).wait()
    pltpu.make_async_copy(dst_v.at[b], out_slice(0), souts[b]).wait()

    @pl.when(wid < N_BLOCKS - N_COMMON)
    def _():
        c0 = pl.multiple_of((N_COMMON + wid) * W_PACK, 128)
        pltpu.sync_copy(tblT_hbm.at[:, pl.ds(c0, W_PACK)], slab_v.at[0])
        transpose_slab(0, W_PACK)
        f0 = pl.multiple_of((N_COMMON + wid) * (W_PACK // 2), 8)
        pltpu.sync_copy(dst_v.at[0], out_hbm.at[pl.ds(f0, W_PACK // 2)])

    @pl.when(wid == 20)
    def _():
        pltpu.sync_copy(tail_hbm, dst_v.at[1, pl.ds(0, 32)])
        pltpu.sync_copy(
            dst_v.at[1, pl.ds(0, 32)],
            out_hbm.at[pl.ds(REM_BASE // 2, 32)],
        )


_pack = functools.partial(
    pl.kernel,
    out_type=jax.ShapeDtypeStruct((VOCAB // 2, EMBED_PAD), jnp.float32),
    mesh=plsc.VectorSubcoreMesh(core_axis_name="c", subcore_axis_name="s"),
    scratch_types=[
        pltpu.VMEM((2, EMBED, W_PACK), jnp.float32),
        pltpu.VMEM((2, W_PACK // 2, EMBED_PAD), jnp.float32),
        pltpu.SemaphoreType.DMA,
        pltpu.SemaphoreType.DMA,
        pltpu.SemaphoreType.DMA,
        pltpu.SemaphoreType.DMA,
    ],
    compiler_params=pltpu.CompilerParams(needs_layout_passes=False),
)(_pack_body)


def _pool_body(xp_hbm, pw_hbm, tbl_hbm, out_hbm, idx_v, pw_v, rows_v,
               pooled_v, sem0, sem1):
    wid = lax.axis_index("s") * NC + lax.axis_index("c")
    base = wid * B_PER_W

    pltpu.sync_copy(xp_hbm.at[pl.ds(wid * IDX_PER_W, IDX_PER_W)], idx_v)
    pltpu.sync_copy(
        pw_hbm.at[pl.ds(wid * PW_PER_W, PW_PER_W)],
        pw_v.at[pl.ds(0, PW_PER_W)],
    )

    sems = (sem0, sem1)

    def idx_slice(i):
        off = pl.multiple_of(i * SEQ_PAD, SEQ_PAD)
        return idx_v.at[pl.ds(off, SEQ)]

    pltpu.async_copy(tbl_hbm.at[idx_slice(0)], rows_v.at[0], sem0)

    neg_inf = jnp.full((LANES,), -jnp.inf, dtype=jnp.float32)

    def outer(g, carry):
        for b in range(2):
            i = g * 2 + b
            pltpu.make_async_copy(
                tbl_hbm.at[idx_slice(0)], rows_v.at[b], sems[b]
            ).wait()

            @pl.when(i + 1 < B_PER_W)
            def _():
                pltpu.async_copy(
                    tbl_hbm.at[idx_slice(i + 1)], rows_v.at[1 - b],
                    sems[1 - b]
                )

            def red(t, accs):
                a = list(accs)
                j0 = t * UNROLL
                w = pw_v[pl.ds(i * PW_PER_ROW + t // 4, LANES)][0]
                s0 = (t % 4) * UNROLL
                for u in range(UNROLL):
                    colbase = ((w >> (s0 + u)) & 1) * EMBED
                    for q in range(QV):
                        a[q] = jnp.maximum(
                            a[q],
                            rows_v[b, j0 + u,
                                   pl.ds(colbase + q * LANES, LANES)],
                        )
                return tuple(a)

            accs = lax.fori_loop(
                0, SEQ // UNROLL, red, (neg_inf,) * QV, unroll=False
            )
            for q in range(QV):
                pooled_v[i, pl.ds(q * LANES, LANES)] = accs[q]
        return carry

    lax.fori_loop(0, B_PER_W // 2, outer, 0, unroll=False)

    pltpu.sync_copy(pooled_v, out_hbm.at[pl.ds(base, B_PER_W)])


_pool = functools.partial(
    pl.kernel,
    out_type=jax.ShapeDtypeStruct((BATCH, EMBED_PAD), jnp.float32),
    mesh=plsc.VectorSubcoreMesh(core_axis_name="c", subcore_axis_name="s"),
    scratch_types=[
        pltpu.VMEM((IDX_PER_W,), jnp.int32),
        pltpu.VMEM((PW_PER_W + LANES,), jnp.int32),
        pltpu.VMEM((2, SEQ, EMBED_PAD), jnp.float32),
        pltpu.VMEM((B_PER_W, EMBED_PAD), jnp.float32),
        pltpu.SemaphoreType.DMA,
        pltpu.SemaphoreType.DMA,
    ],
)(_pool_body)


def _mlp_body(p_ref, w1_ref, b1_ref, w2_ref, b2_ref, o_ref):
    pooled = p_ref[...][:, :EMBED]
    h = jnp.maximum(
        jnp.dot(pooled, w1_ref[...], preferred_element_type=jnp.float32)
        + b1_ref[...],
        0.0,
    )
    o_ref[...] = (
        jnp.dot(h, w2_ref[...], preferred_element_type=jnp.float32)
        + b2_ref[...]
    )


def kernel(x, table, W1, b1, W2, b2):
    xi = x.astype(jnp.int32)
    xp = jnp.pad(xi >> 1, ((0, 0), (0, SEQ_PAD - SEQ)))
    par = jnp.pad((xi & 1).astype(jnp.uint32), ((0, 0), (0, 224 - SEQ)))
    shifts = jnp.arange(32, dtype=jnp.uint32)
    pw = jnp.sum(par.reshape(BATCH, 7, 32) << shifts, axis=-1,
                 dtype=jnp.uint32)
    pw = jnp.pad(pw, ((0, 0), (0, PW_PER_ROW - 7))).astype(jnp.int32)
    tail2 = jnp.reshape(table[N_BLOCKS * W_PACK:], (32, EMBED_PAD))
    tbl2 = _pack(table.T, tail2)
    pooled = _pool(xp.reshape(-1), pw.reshape(-1), tbl2)
    out = pl.pallas_call(
        _mlp_body,
        out_shape=jax.ShapeDtypeStruct((BATCH, NUM_CLASSES), jnp.float32),
    )(pooled, W1, b1.reshape(1, EMBED), W2, b2.reshape(1, NUM_CLASSES))
    return out


# speedup vs baseline: 13.8658x; 2.2558x over previous
import functools

import jax
import jax.numpy as jnp
from jax import lax
from jax.experimental import pallas as pl
from jax.experimental.pallas import tpu as pltpu
from jax.experimental.pallas import tpu_sc as plsc

VOCAB = 1000000
EMBED = 64
EMBED_PAD = 128
NUM_CLASSES = 16
BATCH = 4096
SEQ = 200

NC = 2
NS = 16
NW = NC * NS
B_PER_W = BATCH // NW
LANES = 16
QV = EMBED // LANES
UNROLL = 8
SEQ_PAD = 256
IDX_PER_W = B_PER_W * SEQ_PAD
PW_PER_ROW = 8
PW_PER_W = B_PER_W * PW_PER_ROW


C_PACK = 8192


def _pack_body(in_ref, o_ref):
    x = in_ref[...]
    y = jnp.transpose(x, (1, 0))
    y2 = y.reshape(C_PACK // 2, 2, EMBED)
    o_ref[:, :EMBED] = y2[:, 0, :]
    o_ref[:, EMBED:] = y2[:, 1, :]


def _pack(tblT):
    grid = (VOCAB + C_PACK - 1) // C_PACK
    return pl.pallas_call(
        _pack_body,
        grid=(grid,),
        in_specs=[pl.BlockSpec((EMBED, C_PACK), lambda i: (0, i))],
        out_specs=pl.BlockSpec((C_PACK // 2, EMBED_PAD), lambda i: (i, 0)),
        out_shape=jax.ShapeDtypeStruct((VOCAB // 2, EMBED_PAD), jnp.float32),
    )(tblT)


def _pool_body(xp_hbm, pw_hbm, tbl_hbm, out_hbm, idx_v, pw_v, rows_v,
               pooled_v, sem0, sem1):
    wid = lax.axis_index("s") * NC + lax.axis_index("c")
    base = wid * B_PER_W

    pltpu.sync_copy(xp_hbm.at[pl.ds(wid * IDX_PER_W, IDX_PER_W)], idx_v)
    pltpu.sync_copy(
        pw_hbm.at[pl.ds(wid * PW_PER_W, PW_PER_W)],
        pw_v.at[pl.ds(0, PW_PER_W)],
    )

    sems = (sem0, sem1)

    def idx_slice(i):
        off = pl.multiple_of(i * SEQ_PAD, SEQ_PAD)
        return idx_v.at[pl.ds(off, SEQ)]

    pltpu.async_copy(tbl_hbm.at[idx_slice(0)], rows_v.at[0], sem0)

    neg_inf = jnp.full((LANES,), -jnp.inf, dtype=jnp.float32)

    def outer(g, carry):
        for b in range(2):
            i = g * 2 + b
            pltpu.make_async_copy(
                tbl_hbm.at[idx_slice(0)], rows_v.at[b], sems[b]
            ).wait()

            @pl.when(i + 1 < B_PER_W)
            def _():
                pltpu.async_copy(
                    tbl_hbm.at[idx_slice(i + 1)], rows_v.at[1 - b],
                    sems[1 - b]
                )

            def red(t, accs):
                a = list(accs)
                j0 = t * UNROLL
                w = pw_v[pl.ds(i * PW_PER_ROW + t // 4, LANES)][0]
                s0 = (t % 4) * UNROLL
                for u in range(UNROLL):
                    colbase = ((w >> (s0 + u)) & 1) * EMBED
                    for q in range(QV):
                        a[q] = jnp.maximum(
                            a[q],
                            rows_v[b, j0 + u,
                                   pl.ds(colbase + q * LANES, LANES)],
                        )
                return tuple(a)

            accs = lax.fori_loop(
                0, SEQ // UNROLL, red, (neg_inf,) * QV, unroll=False
            )
            for q in range(QV):
                pooled_v[i, pl.ds(q * LANES, LANES)] = accs[q]
        return carry

    lax.fori_loop(0, B_PER_W // 2, outer, 0, unroll=False)

    pltpu.sync_copy(pooled_v, out_hbm.at[pl.ds(base, B_PER_W)])


_pool = functools.partial(
    pl.kernel,
    out_type=jax.ShapeDtypeStruct((BATCH, EMBED_PAD), jnp.float32),
    mesh=plsc.VectorSubcoreMesh(core_axis_name="c", subcore_axis_name="s"),
    scratch_types=[
        pltpu.VMEM((IDX_PER_W,), jnp.int32),
        pltpu.VMEM((PW_PER_W + LANES,), jnp.int32),
        pltpu.VMEM((2, SEQ, EMBED_PAD), jnp.float32),
        pltpu.VMEM((B_PER_W, EMBED_PAD), jnp.float32),
        pltpu.SemaphoreType.DMA,
        pltpu.SemaphoreType.DMA,
    ],
)(_pool_body)


def _mlp_body(p_ref, w1_ref, b1_ref, w2_ref, b2_ref, o_ref):
    pooled = p_ref[...][:, :EMBED]
    h = jnp.maximum(
        jnp.dot(pooled, w1_ref[...], preferred_element_type=jnp.float32)
        + b1_ref[...],
        0.0,
    )
    o_ref[...] = (
        jnp.dot(h, w2_ref[...], preferred_element_type=jnp.float32)
        + b2_ref[...]
    )


def kernel(x, table, W1, b1, W2, b2):
    xi = x.astype(jnp.int32)
    xp = jnp.pad(xi >> 1, ((0, 0), (0, SEQ_PAD - SEQ)))
    par = jnp.pad((xi & 1).astype(jnp.uint32), ((0, 0), (0, 224 - SEQ)))
    shifts = jnp.arange(32, dtype=jnp.uint32)
    pw = jnp.sum(par.reshape(BATCH, 7, 32) << shifts, axis=-1,
                 dtype=jnp.uint32)
    pw = jnp.pad(pw, ((0, 0), (0, PW_PER_ROW - 7))).astype(jnp.int32)
    tbl2 = _pack(table.T)
    pooled = _pool(xp.reshape(-1), pw.reshape(-1), tbl2)
    out = pl.pallas_call(
        _mlp_body,
        out_shape=jax.ShapeDtypeStruct((BATCH, NUM_CLASSES), jnp.float32),
    )(pooled, W1, b1.reshape(1, EMBED), W2, b2.reshape(1, NUM_CLASSES))
    return out


# speedup vs baseline: 17.3485x; 1.2512x over previous
import functools

import jax
import jax.numpy as jnp
from jax import lax
from jax.experimental import pallas as pl
from jax.experimental.pallas import tpu as pltpu
from jax.experimental.pallas import tpu_sc as plsc

VOCAB = 1000000
EMBED = 64
EMBED_PAD = 128
NUM_CLASSES = 16
BATCH = 4096
SEQ = 200

NC = 2
NS = 16
NW = NC * NS
B_PER_W = BATCH // NW
LANES = 16
QV = EMBED // LANES
UNROLL = 8
SEQ_PAD = 256
IDX_PER_W = B_PER_W * SEQ_PAD
PW_PER_ROW = 8
PW_PER_W = B_PER_W * PW_PER_ROW


C_PACK = 8192
P_BLK = C_PACK // 2
N_GRID = (VOCAB + C_PACK - 1) // C_PACK
N_ROWS = N_GRID * P_BLK


def _pack_body(in_ref, o_ref):
    x = in_ref[...]
    o_ref[:, :EMBED] = jnp.transpose(x[:, :P_BLK], (1, 0))
    o_ref[:, EMBED:] = jnp.transpose(x[:, P_BLK:], (1, 0))


def _pack(tblT):
    return pl.pallas_call(
        _pack_body,
        grid=(N_GRID,),
        in_specs=[pl.BlockSpec((EMBED, C_PACK), lambda i: (0, i))],
        out_specs=pl.BlockSpec((P_BLK, EMBED_PAD), lambda i: (i, 0)),
        out_shape=jax.ShapeDtypeStruct((N_ROWS, EMBED_PAD), jnp.float32),
    )(tblT)


def _pool_body(xp_hbm, pw_hbm, tbl_hbm, out_hbm, idx_v, pw_v, rows_v,
               pooled_v, sem0, sem1):
    wid = lax.axis_index("s") * NC + lax.axis_index("c")
    base = wid * B_PER_W

    pltpu.sync_copy(xp_hbm.at[pl.ds(wid * IDX_PER_W, IDX_PER_W)], idx_v)
    pltpu.sync_copy(
        pw_hbm.at[pl.ds(wid * PW_PER_W, PW_PER_W)],
        pw_v.at[pl.ds(0, PW_PER_W)],
    )

    sems = (sem0, sem1)

    def idx_slice(i):
        off = pl.multiple_of(i * SEQ_PAD, SEQ_PAD)
        return idx_v.at[pl.ds(off, SEQ)]

    pltpu.async_copy(tbl_hbm.at[idx_slice(0)], rows_v.at[0], sem0)

    neg_inf = jnp.full((LANES,), -jnp.inf, dtype=jnp.float32)

    def outer(g, carry):
        for b in range(2):
            i = g * 2 + b
            pltpu.make_async_copy(
                tbl_hbm.at[idx_slice(0)], rows_v.at[b], sems[b]
            ).wait()

            @pl.when(i + 1 < B_PER_W)
            def _():
                pltpu.async_copy(
                    tbl_hbm.at[idx_slice(i + 1)], rows_v.at[1 - b],
                    sems[1 - b]
                )

            def red(t, accs):
                a = list(accs)
                j0 = t * UNROLL
                w = pw_v[pl.ds(i * PW_PER_ROW + t // 4, LANES)][0]
                s0 = (t % 4) * UNROLL
                for u in range(UNROLL):
                    colbase = ((w >> (s0 + u)) & 1) * EMBED
                    for q in range(QV):
                        a[q] = jnp.maximum(
                            a[q],
                            rows_v[b, j0 + u,
                                   pl.ds(colbase + q * LANES, LANES)],
                        )
                return tuple(a)

            accs = lax.fori_loop(
                0, SEQ // UNROLL, red, (neg_inf,) * QV, unroll=False
            )
            for q in range(QV):
                pooled_v[i, pl.ds(q * LANES, LANES)] = accs[q]
        return carry

    lax.fori_loop(0, B_PER_W // 2, outer, 0, unroll=False)

    pltpu.sync_copy(pooled_v, out_hbm.at[pl.ds(base, B_PER_W)])


_pool = functools.partial(
    pl.kernel,
    out_type=jax.ShapeDtypeStruct((BATCH, EMBED_PAD), jnp.float32),
    mesh=plsc.VectorSubcoreMesh(core_axis_name="c", subcore_axis_name="s"),
    scratch_types=[
        pltpu.VMEM((IDX_PER_W,), jnp.int32),
        pltpu.VMEM((PW_PER_W + LANES,), jnp.int32),
        pltpu.VMEM((2, SEQ, EMBED_PAD), jnp.float32),
        pltpu.VMEM((B_PER_W, EMBED_PAD), jnp.float32),
        pltpu.SemaphoreType.DMA,
        pltpu.SemaphoreType.DMA,
    ],
)(_pool_body)


def _mlp_body(p_ref, w1_ref, b1_ref, w2_ref, b2_ref, o_ref):
    pooled = p_ref[...][:, :EMBED]
    h = jnp.maximum(
        jnp.dot(pooled, w1_ref[...], preferred_element_type=jnp.float32)
        + b1_ref[...],
        0.0,
    )
    o_ref[...] = (
        jnp.dot(h, w2_ref[...], preferred_element_type=jnp.float32)
        + b2_ref[...]
    )


def kernel(x, table, W1, b1, W2, b2):
    xi = x.astype(jnp.int32)
    j = xi & (C_PACK - 1)
    xp_full = ((xi >> 13) << 12) | (j & (P_BLK - 1))
    xp = jnp.pad(xp_full, ((0, 0), (0, SEQ_PAD - SEQ)))
    par = jnp.pad((j >> 12).astype(jnp.uint32), ((0, 0), (0, 224 - SEQ)))
    shifts = jnp.arange(32, dtype=jnp.uint32)
    pw = jnp.sum(par.reshape(BATCH, 7, 32) << shifts, axis=-1,
                 dtype=jnp.uint32)
    pw = jnp.pad(pw, ((0, 0), (0, PW_PER_ROW - 7))).astype(jnp.int32)
    tbl2 = _pack(table.T)
    pooled = _pool(xp.reshape(-1), pw.reshape(-1), tbl2)
    out = pl.pallas_call(
        _mlp_body,
        out_shape=jax.ShapeDtypeStruct((BATCH, NUM_CLASSES), jnp.float32),
    )(pooled, W1, b1.reshape(1, EMBED), W2, b2.reshape(1, NUM_CLASSES))
    return out


# speedup vs baseline: 19.6612x; 1.1333x over previous
import functools

import jax
import jax.numpy as jnp
from jax import lax
from jax.experimental import pallas as pl
from jax.experimental.pallas import tpu as pltpu
from jax.experimental.pallas import tpu_sc as plsc

VOCAB = 1000000
EMBED = 64
EMBED_PAD = 128
NUM_CLASSES = 16
BATCH = 4096
SEQ = 200

NC = 2
NS = 16
NW = NC * NS
B_PER_W = BATCH // NW
LANES = 16
QV = EMBED // LANES
UNROLL = 8
SEQ_PAD = 256
IDX_PER_W = B_PER_W * SEQ_PAD
PW_PER_ROW = 8
PW_PER_W = B_PER_W * PW_PER_ROW


C_PACK = 8192
P_BLK = C_PACK // 2
N_GRID = (VOCAB + C_PACK - 1) // C_PACK
N_ROWS = N_GRID * P_BLK


def _pack_body(in_ref, o_ref):
    x = in_ref[...]
    o_ref[:, :EMBED] = jnp.transpose(x[:, :P_BLK], (1, 0))
    o_ref[:, EMBED:] = jnp.transpose(x[:, P_BLK:], (1, 0))


def _pack(tblT):
    return pl.pallas_call(
        _pack_body,
        grid=(N_GRID,),
        in_specs=[pl.BlockSpec((EMBED, C_PACK), lambda i: (0, i))],
        out_specs=pl.BlockSpec((P_BLK, EMBED_PAD), lambda i: (i, 0)),
        out_shape=jax.ShapeDtypeStruct((N_ROWS, EMBED_PAD), jnp.float32),
    )(tblT)


def _pool_body(xp_hbm, pw_hbm, tbl_hbm, out_hbm, idx_v, pw_v, rows_v,
               pooled_v, sem0, sem1, sem2):
    wid = lax.axis_index("s") * NC + lax.axis_index("c")
    base = wid * B_PER_W

    pltpu.sync_copy(xp_hbm.at[pl.ds(wid * IDX_PER_W, IDX_PER_W)], idx_v)
    pltpu.sync_copy(
        pw_hbm.at[pl.ds(wid * PW_PER_W, PW_PER_W)],
        pw_v.at[pl.ds(0, PW_PER_W)],
    )

    sems = (sem0, sem1, sem2)
    NBUF = 3

    def idx_slice(i):
        off = pl.multiple_of(i * SEQ_PAD, SEQ_PAD)
        return idx_v.at[pl.ds(off, SEQ)]

    pltpu.async_copy(tbl_hbm.at[idx_slice(0)], rows_v.at[0], sem0)
    pltpu.async_copy(tbl_hbm.at[idx_slice(1)], rows_v.at[1], sem1)

    neg_inf = jnp.full((LANES,), -jnp.inf, dtype=jnp.float32)

    def do_row(i, b):
        pltpu.make_async_copy(
            tbl_hbm.at[idx_slice(0)], rows_v.at[b], sems[b]
        ).wait()

        @pl.when(i + 2 < B_PER_W)
        def _():
            nb = (b + 2) % NBUF
            pltpu.async_copy(
                tbl_hbm.at[idx_slice(i + 2)], rows_v.at[nb], sems[nb]
            )

        def red(t, accs):
            a = list(accs)
            j0 = t * UNROLL
            w = pw_v[pl.ds(i * PW_PER_ROW + t // 4, LANES)][0]
            s0 = (t % 4) * UNROLL
            for u in range(UNROLL):
                colbase = ((w >> (s0 + u)) & 1) * EMBED
                for q in range(QV):
                    a[q] = jnp.maximum(
                        a[q],
                        rows_v[b, j0 + u,
                               pl.ds(colbase + q * LANES, LANES)],
                    )
            return tuple(a)

        accs = lax.fori_loop(
            0, SEQ // UNROLL, red, (neg_inf,) * QV, unroll=False
        )
        for q in range(QV):
            pooled_v[i, pl.ds(q * LANES, LANES)] = accs[q]

    def outer(g, carry):
        for b in range(NBUF):
            do_row(g * NBUF + b, b)
        return carry

    lax.fori_loop(0, B_PER_W // NBUF, outer, 0, unroll=False)
    for i in range(B_PER_W - B_PER_W % NBUF, B_PER_W):
        do_row(i, i % NBUF)

    pltpu.sync_copy(pooled_v, out_hbm.at[pl.ds(base, B_PER_W)])


_pool = functools.partial(
    pl.kernel,
    out_type=jax.ShapeDtypeStruct((BATCH, EMBED_PAD), jnp.float32),
    mesh=plsc.VectorSubcoreMesh(core_axis_name="c", subcore_axis_name="s"),
    scratch_types=[
        pltpu.VMEM((IDX_PER_W,), jnp.int32),
        pltpu.VMEM((PW_PER_W + LANES,), jnp.int32),
        pltpu.VMEM((3, SEQ, EMBED_PAD), jnp.float32),
        pltpu.VMEM((B_PER_W, EMBED_PAD), jnp.float32),
        pltpu.SemaphoreType.DMA,
        pltpu.SemaphoreType.DMA,
        pltpu.SemaphoreType.DMA,
    ],
)(_pool_body)


def _mlp_body(p_ref, w1_ref, b1_ref, w2_ref, b2_ref, o_ref):
    pooled = p_ref[...][:, :EMBED]
    h = jnp.maximum(
        jnp.dot(pooled, w1_ref[...], preferred_element_type=jnp.float32)
        + b1_ref[...],
        0.0,
    )
    o_ref[...] = (
        jnp.dot(h, w2_ref[...], preferred_element_type=jnp.float32)
        + b2_ref[...]
    )


def kernel(x, table, W1, b1, W2, b2):
    xi = x.astype(jnp.int32)
    j = xi & (C_PACK - 1)
    xp_full = ((xi >> 13) << 12) | (j & (P_BLK - 1))
    xp = jnp.pad(xp_full, ((0, 0), (0, SEQ_PAD - SEQ)))
    par = jnp.pad((j >> 12).astype(jnp.uint32), ((0, 0), (0, 224 - SEQ)))
    shifts = jnp.arange(32, dtype=jnp.uint32)
    pw = jnp.sum(par.reshape(BATCH, 7, 32) << shifts, axis=-1,
                 dtype=jnp.uint32)
    pw = jnp.pad(pw, ((0, 0), (0, PW_PER_ROW - 7))).astype(jnp.int32)
    tbl2 = _pack(table.T)
    pooled = _pool(xp.reshape(-1), pw.reshape(-1), tbl2)
    out = pl.pallas_call(
        _mlp_body,
        out_shape=jax.ShapeDtypeStruct((BATCH, NUM_CLASSES), jnp.float32),
    )(pooled, W1, b1.reshape(1, EMBED), W2, b2.reshape(1, NUM_CLASSES))
    return out
